# baseline (speedup 1.0000x reference)
import functools

import jax
import jax.numpy as jnp
from jax import lax
from jax.experimental import pallas as pl
from jax.experimental.pallas import tpu as pltpu
from jax.experimental.pallas import tpu_sc as plsc

_NCORES = 2
_NSUB = 16
_NW = _NCORES * _NSUB
_L = 16

_NB = 5
_NCF = 5
_NK = 16
_D = 64
_TOK = 1 + _NB + _NCF + _NK

_CH = 2048
_UNROLL = 8


@functools.partial(jax.jit, static_argnames=("batch", "vocab"))
def _tokenize(batch, vocab, bidx_t, cidx_t, cval_ti, bin_tbl, w_tbl,
              b_tbl, m_tbl, cls_v, cat_t):
    nch = batch // _CH
    npair = nch // 2
    grp_u = _CH // _L // _UNROLL
    d_half = _D // 2
    bc_per_w = (_NB * _D) // _NW
    cls_per_w = _D // _NW

    mesh = plsc.VectorSubcoreMesh(
        core_axis_name="c", subcore_axis_name="s",
        num_cores=_NCORES, num_subcores=_NSUB)

    @functools.partial(
        pl.kernel,
        out_type=jax.ShapeDtypeStruct((_TOK, _D, batch), jnp.float32),
        mesh=mesh,
        scratch_types=[
            pltpu.VMEM((vocab,), jnp.float32),
            pltpu.VMEM((batch,), jnp.int32),
            pltpu.VMEM((2 * _CH,), jnp.int32),
            pltpu.VMEM((2 * _CH,), jnp.float32),
            pltpu.VMEM((_NB * 2 * _D,), jnp.float32),
            pltpu.VMEM((_NCF * _D,), jnp.float32),
            pltpu.VMEM((_NCF * _D,), jnp.float32),
            pltpu.VMEM((_NCF * _D,), jnp.float32),
            pltpu.VMEM((_D,), jnp.float32),
            pltpu.SemaphoreType.DMA,
            pltpu.SemaphoreType.DMA,
            pltpu.SemaphoreType.DMA,
            pltpu.SemaphoreType.DMA,
            pltpu.SemaphoreType.DMA,
        ],
        compiler_params=pltpu.CompilerParams(
            use_tc_tiling_on_sc=True, needs_layout_passes=False),
    )
    def tok_kernel(bidx_h, cidx_h, cval_h, bintbl_h, wtbl_h, btbl_h,
                   mtbl_h, cls_h, cat_h, out_h,
                   tblrow, colbuf, dcol, outbuf, binv, wv, bv, mv, clsv,
                   sem_row, sem_out0, sem_out1, sem_col0, sem_col1):
        w = lax.axis_index("s") * _NCORES + lax.axis_index("c")
        f_cat = w // 2
        d0 = (w % 2) * d_half
        sem_out = (sem_out0, sem_out1)
        sem_col = (sem_col0, sem_col1)

        pltpu.sync_copy(bintbl_h, binv)
        pltpu.sync_copy(wtbl_h, wv)
        pltpu.sync_copy(btbl_h, bv)
        pltpu.sync_copy(mtbl_h, mv)
        pltpu.sync_copy(cls_h, clsv)
        pltpu.sync_copy(cidx_h.at[f_cat, :], colbuf)

        def splat(tbl, pos):
            return plsc.load_gather(tbl, [jnp.full((_L,), pos, jnp.int32)])

        def drain_out(p):
            pltpu.make_async_copy(
                out_h.at[0, 0, pl.ds(0, _CH)],
                outbuf.at[pl.ds(p * _CH, _CH)], sem_out[p]).wait()

        def drain_col(p):
            pltpu.make_async_copy(
                bidx_h.at[0, pl.ds(0, _CH)],
                dcol.at[pl.ds(p * _CH, _CH)], sem_col[p]).wait()

        def start_out(p, tok, d, c):
            pltpu.make_async_copy(
                outbuf.at[pl.ds(p * _CH, _CH)],
                out_h.at[tok, d, pl.ds(c * _CH, _CH)], sem_out[p]).start()

        def start_col(src_h, f, c, pc):
            pltpu.make_async_copy(
                src_h.at[f, pl.ds(c * _CH, _CH)],
                dcol.at[pl.ds(pc * _CH, _CH)], sem_col[pc]).start()

        def start_row(d):
            pltpu.make_async_copy(cat_h.at[f_cat, d, :], tblrow,
                                  sem_row).start()

        def drain_row():
            pltpu.make_async_copy(cat_h.at[f_cat, 0, :], tblrow,
                                  sem_row).wait()

        start_row(d0)

        def fill_chunk(p, c, make_group):
            def body(j, _):
                for u in range(_UNROLL):
                    g = j * _UNROLL + u
                    make_group(c * _CH + g * _L, p * _CH + g * _L)
                return 0
            lax.fori_loop(0, grp_u, body, 0)

        def emit_dense(tok, d, make_group, src_h, f_src):
            start_col(src_h, f_src, 0, 0)

            def pair(t, _):
                for p in range(2):
                    c = 2 * t + p
                    drain_col(p)
                    start_col(src_h, f_src, jnp.minimum(c + 1, nch - 1),
                              (p + 1) % 2)
                    drain_out(p)

                    def mk(src_off, dst_off, _p=p):
                        make_group(_p * _CH + (src_off - c * _CH), dst_off)
                    fill_chunk(p, c, mk)
                    start_out(p, tok, d, c)
                return 0
            lax.fori_loop(0, npair, pair, 0)
            drain_col(0)

        def cls_rows(i):
            d = w * cls_per_w + i
            s = splat(clsv, d)

            def pair(t, _):
                for p in range(2):
                    c = 2 * t + p
                    drain_out(p)

                    def mk(src_off, dst_off):
                        outbuf[pl.ds(dst_off, _L)] = s
                    fill_chunk(p, c, mk)
                    start_out(p, 0, d, c)
                return 0
            lax.fori_loop(0, npair, pair, 0)

        def bin_rows(i):
            r = w * bc_per_w + i
            f = r // _D
            d = r % _D
            e0 = splat(binv, f * 2 * _D + d)
            df = splat(binv, f * 2 * _D + _D + d) - e0

            def mk(src_off, dst_off):
                x = dcol[pl.ds(src_off, _L)]
                outbuf[pl.ds(dst_off, _L)] = (
                    e0 + x.astype(jnp.float32) * df)
            emit_dense(1 + f, d, mk, bidx_h, f)

        def cont_rows(i):
            r = w * bc_per_w + i
            f = r // _D
            d = r % _D
            ws = splat(wv, f * _D + d)
            bs = splat(bv, f * _D + d)
            ms = splat(mv, f * _D + d)

            def mk(src_off, dst_off):
                v = plsc.bitcast(dcol[pl.ds(src_off, _L)], jnp.float32)
                miss = v != v
                vz = jnp.where(miss, jnp.float32(0), v)
                outbuf[pl.ds(dst_off, _L)] = jnp.where(
                    miss, ms, vz * ws + bs)
            emit_dense(1 + _NB + f, d, mk, cval_h, f)

        def cat_iter(dd, _):
            d = d0 + dd
            drain_row()

            def pair(t, _):
                for p in range(2):
                    c = 2 * t + p
                    lax.cond(jnp.logical_or(dd > 0, t > 0),
                             lambda _p=p: drain_out(_p), lambda: None)

                    def mk(src_off, dst_off):
                        idx = colbuf[pl.ds(src_off, _L)]
                        outbuf[pl.ds(dst_off, _L)] = (
                            plsc.load_gather(tblrow, [idx]))
                    fill_chunk(p, c, mk)
                    start_out(p, 1 + _NB + _NCF + f_cat, d, c)
                return 0
            lax.fori_loop(0, npair, pair, 0)

            start_row(jnp.minimum(d0 + dd + 1, _D - 1))

            lax.cond(
                dd < cls_per_w,
                lambda: cls_rows(dd),
                lambda: lax.cond(
                    dd < cls_per_w + bc_per_w,
                    lambda: bin_rows(dd - cls_per_w),
                    lambda: lax.cond(
                        dd < cls_per_w + 2 * bc_per_w,
                        lambda: cont_rows(dd - cls_per_w - bc_per_w),
                        lambda: None)))
            return 0

        lax.fori_loop(0, d_half, cat_iter, 0)

        drain_row()
        drain_out(0)
        drain_out(1)

    return tok_kernel(bidx_t, cidx_t, cval_ti, bin_tbl, w_tbl, b_tbl,
                      m_tbl, cls_v, cat_t)


def kernel(bin_idx, cat_idx, cont_vals, bin_emb, cont_w, cont_b, cont_mask,
           cat_emb, cls):
    batch = bin_idx.shape[0]
    vocab = cat_emb.shape[1]
    cat_t = cat_emb.transpose(0, 2, 1)
    cidx_t = cat_idx.T
    bidx_t = bin_idx.T
    cval_ti = lax.bitcast_convert_type(cont_vals.T, jnp.int32)
    out = _tokenize(
        batch, vocab, bidx_t, cidx_t, cval_ti,
        bin_emb.reshape(-1), cont_w.reshape(-1), cont_b.reshape(-1),
        cont_mask.reshape(-1), cls, cat_t)
    return out.transpose(2, 0, 1)


# speedup vs baseline: 1.4534x; 1.4534x over previous
import functools

import jax
import jax.numpy as jnp
from jax import lax
from jax.experimental import pallas as pl
from jax.experimental.pallas import tpu as pltpu
from jax.experimental.pallas import tpu_sc as plsc

_NCORES = 2
_NSUB = 16
_NW = _NCORES * _NSUB
_L = 16

_NB = 5
_NCF = 5
_NK = 16
_D = 64
_TOK = 1 + _NB + _NCF + _NK

_CH = 4096
_UNROLL = 8


@functools.partial(jax.jit, static_argnames=("batch", "vocab"))
def _tokenize(batch, vocab, bidx_t, cidx_t, cval_ti, bin_tbl, w_tbl,
              b_tbl, m_tbl, cls_v, cat_t):
    nch = batch // _CH
    grp_u = _CH // _L // _UNROLL
    d_half = _D // 2
    bc_per_w = (_NB * _D) // _NW
    cls_per_w = _D // _NW

    mesh = plsc.VectorSubcoreMesh(
        core_axis_name="c", subcore_axis_name="s",
        num_cores=_NCORES, num_subcores=_NSUB)

    @functools.partial(
        pl.kernel,
        out_type=jax.ShapeDtypeStruct((_TOK, _D, batch), jnp.float32),
        mesh=mesh,
        scratch_types=[
            pltpu.VMEM((vocab,), jnp.float32),
            pltpu.VMEM((batch,), jnp.int32),
            pltpu.VMEM((2 * _CH,), jnp.float32),
            pltpu.VMEM((_NB * 2 * _D,), jnp.float32),
            pltpu.VMEM((_NCF * _D,), jnp.float32),
            pltpu.VMEM((_NCF * _D,), jnp.float32),
            pltpu.VMEM((_NCF * _D,), jnp.float32),
            pltpu.VMEM((_D,), jnp.float32),
            pltpu.SemaphoreType.DMA,
            pltpu.SemaphoreType.DMA,
            pltpu.SemaphoreType.DMA,
        ],
        compiler_params=pltpu.CompilerParams(
            use_tc_tiling_on_sc=True, needs_layout_passes=False),
    )
    def tok_kernel(bidx_h, cidx_h, cval_h, bintbl_h, wtbl_h, btbl_h,
                   mtbl_h, cls_h, cat_h, out_h,
                   tblrow, colbuf, outbuf, binv, wv, bv, mv, clsv,
                   sem_row, sem_out0, sem_out1):
        w = lax.axis_index("s") * _NCORES + lax.axis_index("c")
        f_cat = w // 2
        d0 = (w % 2) * d_half
        sem_out = (sem_out0, sem_out1)

        pltpu.make_async_copy(cat_h.at[f_cat, d0, :], tblrow,
                              sem_row).start()

        pltpu.sync_copy(bintbl_h, binv)
        pltpu.sync_copy(wtbl_h, wv)
        pltpu.sync_copy(btbl_h, bv)
        pltpu.sync_copy(mtbl_h, mv)
        pltpu.sync_copy(cls_h, clsv)

        def splat(tbl, pos):
            return plsc.load_gather(tbl, [jnp.full((_L,), pos, jnp.int32)])

        def drain_out(p):
            pltpu.make_async_copy(
                out_h.at[0, 0, pl.ds(0, _CH)],
                outbuf.at[pl.ds(p * _CH, _CH)], sem_out[p]).wait()

        def start_out(p, tok, d, c):
            pltpu.make_async_copy(
                outbuf.at[pl.ds(p * _CH, _CH)],
                out_h.at[tok, d, pl.ds(c * _CH, _CH)], sem_out[p]).start()

        def emit_chunk(p, c, tok, d, make_group, skip_wait=False):
            if not skip_wait:
                drain_out(p)

            def body(j, _):
                for u in range(_UNROLL):
                    g = j * _UNROLL + u
                    make_group(c * _CH + g * _L, p * _CH + g * _L)
                return 0
            lax.fori_loop(0, grp_u, body, 0)
            start_out(p, tok, d, c)

        first_fill = [True, True]

        def emit_row(tok, d, make_group):
            for c in range(nch):
                p = c % 2
                emit_chunk(p, c, tok, d, make_group,
                           skip_wait=first_fill[p])
                first_fill[p] = False

        for i in range(cls_per_w):
            d = w * cls_per_w + i
            s = splat(clsv, d)

            def mk_cls(src_off, dst_off):
                outbuf[pl.ds(dst_off, _L)] = s
            emit_row(0, d, mk_cls)

        for i in range(bc_per_w):
            r = w * bc_per_w + i
            f = r // _D
            d = r % _D
            pltpu.sync_copy(bidx_h.at[f, :], colbuf)
            e0 = splat(binv, f * 2 * _D + d)
            df = splat(binv, f * 2 * _D + _D + d) - e0

            def mk_bin(src_off, dst_off, e0=e0, df=df):
                x = colbuf[pl.ds(src_off, _L)]
                outbuf[pl.ds(dst_off, _L)] = (
                    e0 + x.astype(jnp.float32) * df)
            emit_row(1 + f, d, mk_bin)

        for i in range(bc_per_w):
            r = w * bc_per_w + i
            f = r // _D
            d = r % _D
            pltpu.sync_copy(cval_h.at[f, :], colbuf)
            ws = splat(wv, f * _D + d)
            bs = splat(bv, f * _D + d)
            ms = splat(mv, f * _D + d)

            def mk_cont(src_off, dst_off, ws=ws, bs=bs, ms=ms):
                v = plsc.bitcast(colbuf[pl.ds(src_off, _L)], jnp.float32)
                miss = v != v
                vz = jnp.where(miss, jnp.float32(0), v)
                outbuf[pl.ds(dst_off, _L)] = jnp.where(
                    miss, ms, vz * ws + bs)
            emit_row(1 + _NB + f, d, mk_cont)

        pltpu.sync_copy(cidx_h.at[f_cat, :], colbuf)

        def cat_iter(dd, _):
            d = d0 + dd
            pltpu.make_async_copy(cat_h.at[f_cat, 0, :], tblrow,
                                  sem_row).wait()
            for c in range(nch):
                p = c % 2

                def mk_cat(src_off, dst_off):
                    idx = colbuf[pl.ds(src_off, _L)]
                    outbuf[pl.ds(dst_off, _L)] = (
                        plsc.load_gather(tblrow, [idx]))
                emit_chunk(p, c, 1 + _NB + _NCF + f_cat, d, mk_cat)
            pltpu.make_async_copy(
                cat_h.at[f_cat, jnp.minimum(d0 + dd + 1, _D - 1), :],
                tblrow, sem_row).start()
            return 0

        lax.fori_loop(0, d_half, cat_iter, 0)

        pltpu.make_async_copy(cat_h.at[f_cat, 0, :], tblrow, sem_row).wait()
        drain_out(0)
        drain_out(1)

    return tok_kernel(bidx_t, cidx_t, cval_ti, bin_tbl, w_tbl, b_tbl,
                      m_tbl, cls_v, cat_t)


def kernel(bin_idx, cat_idx, cont_vals, bin_emb, cont_w, cont_b, cont_mask,
           cat_emb, cls):
    batch = bin_idx.shape[0]
    vocab = cat_emb.shape[1]
    cat_t = cat_emb.transpose(0, 2, 1)
    cidx_t = cat_idx.T
    bidx_t = bin_idx.T
    cval_ti = lax.bitcast_convert_type(cont_vals.T, jnp.int32)
    out = _tokenize(
        batch, vocab, bidx_t, cidx_t, cval_ti,
        bin_emb.reshape(-1), cont_w.reshape(-1), cont_b.reshape(-1),
        cont_mask.reshape(-1), cls, cat_t)
    return out.transpose(2, 0, 1)


# speedup vs baseline: 1.5426x; 1.0614x over previous
import functools

import jax
import jax.numpy as jnp
from jax import lax
from jax.experimental import pallas as pl
from jax.experimental.pallas import tpu as pltpu
from jax.experimental.pallas import tpu_sc as plsc

_NCORES = 2
_NSUB = 16
_NW = _NCORES * _NSUB
_L = 16

_NB = 5
_NCF = 5
_NK = 16
_D = 64
_TOK = 1 + _NB + _NCF + _NK

_CH = 4096
_UNROLL = 8


@functools.partial(jax.jit, static_argnames=("batch", "vocab"))
def _tokenize(batch, vocab, bidx_t, cidx_t, cval_ti, bin_tbl, w_tbl,
              b_tbl, m_tbl, cls_v, cat_t):
    nch = batch // _CH
    grp_u = _CH // _L // _UNROLL
    d_half = _D // 2
    bc_per_w = (_NB * _D) // _NW
    cls_per_w = _D // _NW

    mesh = plsc.VectorSubcoreMesh(
        core_axis_name="c", subcore_axis_name="s",
        num_cores=_NCORES, num_subcores=_NSUB)

    @functools.partial(
        pl.kernel,
        out_type=jax.ShapeDtypeStruct((_TOK, _D, batch), jnp.float32),
        mesh=mesh,
        scratch_types=[
            pltpu.VMEM((vocab,), jnp.float32),
            pltpu.VMEM((batch,), jnp.int32),
            pltpu.VMEM((_CH,), jnp.int32),
            pltpu.VMEM((2 * _CH,), jnp.float32),
            pltpu.VMEM((_NB * 2 * _D,), jnp.float32),
            pltpu.VMEM((_NCF * _D,), jnp.float32),
            pltpu.VMEM((_NCF * _D,), jnp.float32),
            pltpu.VMEM((_NCF * _D,), jnp.float32),
            pltpu.VMEM((_D,), jnp.float32),
            pltpu.SemaphoreType.DMA,
            pltpu.SemaphoreType.DMA,
            pltpu.SemaphoreType.DMA,
        ],
        compiler_params=pltpu.CompilerParams(
            use_tc_tiling_on_sc=True, needs_layout_passes=False),
    )
    def tok_kernel(bidx_h, cidx_h, cval_h, bintbl_h, wtbl_h, btbl_h,
                   mtbl_h, cls_h, cat_h, out_h,
                   tblrow, colbuf, dcol, outbuf, binv, wv, bv, mv, clsv,
                   sem_row, sem_out0, sem_out1):
        w = lax.axis_index("s") * _NCORES + lax.axis_index("c")
        f_cat = w // 2
        d0 = (w % 2) * d_half
        sem_out = (sem_out0, sem_out1)

        pltpu.make_async_copy(cat_h.at[f_cat, d0, :], tblrow,
                              sem_row).start()

        pltpu.sync_copy(bintbl_h, binv)
        pltpu.sync_copy(wtbl_h, wv)
        pltpu.sync_copy(btbl_h, bv)
        pltpu.sync_copy(mtbl_h, mv)
        pltpu.sync_copy(cls_h, clsv)

        def splat(tbl, pos):
            return plsc.load_gather(tbl, [jnp.full((_L,), pos, jnp.int32)])

        def drain_out(p):
            pltpu.make_async_copy(
                out_h.at[0, 0, pl.ds(0, _CH)],
                outbuf.at[pl.ds(p * _CH, _CH)], sem_out[p]).wait()

        def start_out(p, tok, d, c):
            pltpu.make_async_copy(
                outbuf.at[pl.ds(p * _CH, _CH)],
                out_h.at[tok, d, pl.ds(c * _CH, _CH)], sem_out[p]).start()

        def emit_chunk(p, c, tok, d, make_group, skip_wait=False):
            if not skip_wait:
                drain_out(p)

            def body(j, _):
                for u in range(_UNROLL):
                    loc = (j * _UNROLL + u) * _L
                    make_group(c * _CH + loc, loc, p * _CH + loc)
                return 0
            lax.fori_loop(0, grp_u, body, 0)
            start_out(p, tok, d, c)

        def dense_row(tok, d, make_group, src_h, f_src):
            for c in range(nch):
                p = c % 2
                pltpu.sync_copy(src_h.at[f_src, pl.ds(c * _CH, _CH)], dcol)
                emit_chunk(p, c, tok, d, make_group)

        def cls_rows(i):
            d = w * cls_per_w + i
            s = splat(clsv, d)

            def mk_cls(src_off, loc_off, dst_off):
                outbuf[pl.ds(dst_off, _L)] = s
            for c in range(nch):
                emit_chunk(c % 2, c, 0, d, mk_cls)

        def bin_rows(i):
            r = w * bc_per_w + i
            f = r // _D
            d = r % _D
            e0 = splat(binv, f * 2 * _D + d)
            df = splat(binv, f * 2 * _D + _D + d) - e0

            def mk_bin(src_off, loc_off, dst_off):
                x = dcol[pl.ds(loc_off, _L)]
                outbuf[pl.ds(dst_off, _L)] = (
                    e0 + x.astype(jnp.float32) * df)
            dense_row(1 + f, d, mk_bin, bidx_h, f)

        def cont_rows(i):
            r = w * bc_per_w + i
            f = r // _D
            d = r % _D
            ws = splat(wv, f * _D + d)
            bs = splat(bv, f * _D + d)
            ms = splat(mv, f * _D + d)

            def mk_cont(src_off, loc_off, dst_off):
                v = plsc.bitcast(dcol[pl.ds(loc_off, _L)], jnp.float32)
                miss = v != v
                vz = jnp.where(miss, jnp.float32(0), v)
                outbuf[pl.ds(dst_off, _L)] = jnp.where(
                    miss, ms, vz * ws + bs)
            dense_row(1 + _NB + f, d, mk_cont, cval_h, f)

        pltpu.sync_copy(cidx_h.at[f_cat, :], colbuf)

        def mk_cat(src_off, loc_off, dst_off):
            idx = colbuf[pl.ds(src_off, _L)]
            outbuf[pl.ds(dst_off, _L)] = plsc.load_gather(tblrow, [idx])

        def cat_gather(dd, skip_first):
            d = d0 + dd
            pltpu.make_async_copy(cat_h.at[f_cat, 0, :], tblrow,
                                  sem_row).wait()
            for c in range(nch):
                emit_chunk(c % 2, c, 1 + _NB + _NCF + f_cat, d, mk_cat,
                           skip_wait=(skip_first and c < 2))
            pltpu.make_async_copy(
                cat_h.at[f_cat, jnp.minimum(d0 + dd + 1, _D - 1), :],
                tblrow, sem_row).start()

        cat_gather(0, True)
        cls_rows(0)

        def cat_iter(dd, _):
            cat_gather(dd, False)
            lax.cond(
                dd < cls_per_w,
                lambda: cls_rows(dd),
                lambda: lax.cond(
                    dd < cls_per_w + bc_per_w,
                    lambda: bin_rows(dd - cls_per_w),
                    lambda: lax.cond(
                        dd < cls_per_w + 2 * bc_per_w,
                        lambda: cont_rows(dd - cls_per_w - bc_per_w),
                        lambda: None)))
            return 0

        lax.fori_loop(1, d_half, cat_iter, 0)

        pltpu.make_async_copy(cat_h.at[f_cat, 0, :], tblrow, sem_row).wait()
        drain_out(0)
        drain_out(1)

    return tok_kernel(bidx_t, cidx_t, cval_ti, bin_tbl, w_tbl, b_tbl,
                      m_tbl, cls_v, cat_t)


def kernel(bin_idx, cat_idx, cont_vals, bin_emb, cont_w, cont_b, cont_mask,
           cat_emb, cls):
    batch = bin_idx.shape[0]
    vocab = cat_emb.shape[1]
    cat_t = cat_emb.transpose(0, 2, 1)
    cidx_t = cat_idx.T
    bidx_t = bin_idx.T
    cval_ti = lax.bitcast_convert_type(cont_vals.T, jnp.int32)
    out = _tokenize(
        batch, vocab, bidx_t, cidx_t, cval_ti,
        bin_emb.reshape(-1), cont_w.reshape(-1), cont_b.reshape(-1),
        cont_mask.reshape(-1), cls, cat_t)
    return out.transpose(2, 0, 1)


# speedup vs baseline: 1.5430x; 1.0003x over previous
import functools

import jax
import jax.numpy as jnp
from jax import lax
from jax.experimental import pallas as pl
from jax.experimental.pallas import tpu as pltpu
from jax.experimental.pallas import tpu_sc as plsc

_NCORES = 2
_NSUB = 16
_NW = _NCORES * _NSUB
_L = 16

_NB = 5
_NCF = 5
_NK = 16
_D = 64
_TOK = 1 + _NB + _NCF + _NK

_CH = 4096
_UNROLL = 16


@functools.partial(jax.jit, static_argnames=("batch", "vocab"))
def _tokenize(batch, vocab, bidx_t, cidx_t, cval_ti, bin_tbl, w_tbl,
              b_tbl, m_tbl, cls_v, cat_t):
    nch = batch // _CH
    grp_u = _CH // _L // _UNROLL
    d_half = _D // 2
    bc_per_w = (_NB * _D) // _NW
    cls_per_w = _D // _NW

    mesh = plsc.VectorSubcoreMesh(
        core_axis_name="c", subcore_axis_name="s",
        num_cores=_NCORES, num_subcores=_NSUB)

    @functools.partial(
        pl.kernel,
        out_type=jax.ShapeDtypeStruct((_TOK, _D, batch), jnp.float32),
        mesh=mesh,
        scratch_types=[
            pltpu.VMEM((vocab,), jnp.float32),
            pltpu.VMEM((batch,), jnp.int32),
            pltpu.VMEM((_CH,), jnp.int32),
            pltpu.VMEM((2 * _CH,), jnp.float32),
            pltpu.VMEM((_NB * 2 * _D,), jnp.float32),
            pltpu.VMEM((_NCF * _D,), jnp.float32),
            pltpu.VMEM((_NCF * _D,), jnp.float32),
            pltpu.VMEM((_NCF * _D,), jnp.float32),
            pltpu.VMEM((_D,), jnp.float32),
            pltpu.SemaphoreType.DMA,
            pltpu.SemaphoreType.DMA,
            pltpu.SemaphoreType.DMA,
        ],
        compiler_params=pltpu.CompilerParams(
            use_tc_tiling_on_sc=True, needs_layout_passes=False),
    )
    def tok_kernel(bidx_h, cidx_h, cval_h, bintbl_h, wtbl_h, btbl_h,
                   mtbl_h, cls_h, cat_h, out_h,
                   tblrow, colbuf, dcol, outbuf, binv, wv, bv, mv, clsv,
                   sem_row, sem_out0, sem_out1):
        w = lax.axis_index("s") * _NCORES + lax.axis_index("c")
        f_cat = w // 2
        d0 = (w % 2) * d_half
        sem_out = (sem_out0, sem_out1)

        pltpu.make_async_copy(cat_h.at[f_cat, d0, :], tblrow,
                              sem_row).start()

        pltpu.sync_copy(bintbl_h, binv)
        pltpu.sync_copy(wtbl_h, wv)
        pltpu.sync_copy(btbl_h, bv)
        pltpu.sync_copy(mtbl_h, mv)
        pltpu.sync_copy(cls_h, clsv)

        def splat(tbl, pos):
            return plsc.load_gather(tbl, [jnp.full((_L,), pos, jnp.int32)])

        def drain_out(p):
            pltpu.make_async_copy(
                out_h.at[0, 0, pl.ds(0, _CH)],
                outbuf.at[pl.ds(p * _CH, _CH)], sem_out[p]).wait()

        def start_out(p, tok, d, c):
            pltpu.make_async_copy(
                outbuf.at[pl.ds(p * _CH, _CH)],
                out_h.at[tok, d, pl.ds(c * _CH, _CH)], sem_out[p]).start()

        def emit_chunk(p, c, tok, d, make_group, skip_wait=False):
            if not skip_wait:
                drain_out(p)

            def body(j, _):
                for u in range(_UNROLL):
                    loc = (j * _UNROLL + u) * _L
                    make_group(c * _CH + loc, loc, p * _CH + loc)
                return 0
            lax.fori_loop(0, grp_u, body, 0)
            start_out(p, tok, d, c)

        def dense_row(tok, d, make_group, src_h, f_src):
            for c in range(nch):
                p = c % 2
                pltpu.sync_copy(src_h.at[f_src, pl.ds(c * _CH, _CH)], dcol)
                emit_chunk(p, c, tok, d, make_group)

        def cls_rows(i):
            d = w * cls_per_w + i
            s = splat(clsv, d)

            def mk_cls(src_off, loc_off, dst_off):
                outbuf[pl.ds(dst_off, _L)] = s
            for c in range(nch):
                emit_chunk(c % 2, c, 0, d, mk_cls)

        def bin_rows(i):
            r = w * bc_per_w + i
            f = r // _D
            d = r % _D
            e0 = splat(binv, f * 2 * _D + d)
            df = splat(binv, f * 2 * _D + _D + d) - e0

            def mk_bin(src_off, loc_off, dst_off):
                x = dcol[pl.ds(loc_off, _L)]
                outbuf[pl.ds(dst_off, _L)] = (
                    e0 + x.astype(jnp.float32) * df)
            dense_row(1 + f, d, mk_bin, bidx_h, f)

        def cont_rows(i):
            r = w * bc_per_w + i
            f = r // _D
            d = r % _D
            ws = splat(wv, f * _D + d)
            bs = splat(bv, f * _D + d)
            ms = splat(mv, f * _D + d)

            def mk_cont(src_off, loc_off, dst_off):
                v = plsc.bitcast(dcol[pl.ds(loc_off, _L)], jnp.float32)
                miss = v != v
                vz = jnp.where(miss, jnp.float32(0), v)
                outbuf[pl.ds(dst_off, _L)] = jnp.where(
                    miss, ms, vz * ws + bs)
            dense_row(1 + _NB + f, d, mk_cont, cval_h, f)

        pltpu.sync_copy(cidx_h.at[f_cat, :], colbuf)

        def mk_cat(src_off, loc_off, dst_off):
            idx = colbuf[pl.ds(src_off, _L)]
            outbuf[pl.ds(dst_off, _L)] = plsc.load_gather(tblrow, [idx])

        def cat_gather(dd, skip_first):
            d = d0 + dd
            pltpu.make_async_copy(cat_h.at[f_cat, 0, :], tblrow,
                                  sem_row).wait()
            for c in range(nch):
                emit_chunk(c % 2, c, 1 + _NB + _NCF + f_cat, d, mk_cat,
                           skip_wait=(skip_first and c < 2))
            pltpu.make_async_copy(
                cat_h.at[f_cat, jnp.minimum(d0 + dd + 1, _D - 1), :],
                tblrow, sem_row).start()

        cat_gather(0, True)
        cls_rows(0)

        def cat_iter(dd, _):
            cat_gather(dd, False)
            lax.cond(
                dd < cls_per_w,
                lambda: cls_rows(dd),
                lambda: lax.cond(
                    dd < cls_per_w + bc_per_w,
                    lambda: bin_rows(dd - cls_per_w),
                    lambda: lax.cond(
                        dd < cls_per_w + 2 * bc_per_w,
                        lambda: cont_rows(dd - cls_per_w - bc_per_w),
                        lambda: None)))
            return 0

        lax.fori_loop(1, d_half, cat_iter, 0)

        pltpu.make_async_copy(cat_h.at[f_cat, 0, :], tblrow, sem_row).wait()
        drain_out(0)
        drain_out(1)

    return tok_kernel(bidx_t, cidx_t, cval_ti, bin_tbl, w_tbl, b_tbl,
                      m_tbl, cls_v, cat_t)


def kernel(bin_idx, cat_idx, cont_vals, bin_emb, cont_w, cont_b, cont_mask,
           cat_emb, cls):
    batch = bin_idx.shape[0]
    vocab = cat_emb.shape[1]
    cat_t = cat_emb.transpose(0, 2, 1)
    cidx_t = cat_idx.T
    bidx_t = bin_idx.T
    cval_ti = lax.bitcast_convert_type(cont_vals.T, jnp.int32)
    out = _tokenize(
        batch, vocab, bidx_t, cidx_t, cval_ti,
        bin_emb.reshape(-1), cont_w.reshape(-1), cont_b.reshape(-1),
        cont_mask.reshape(-1), cls, cat_t)
    return out.transpose(2, 0, 1)
